# baseline (speedup 1.0000x reference)
import functools

import jax
import jax.numpy as jnp
from jax import lax
from jax.experimental import pallas as pl
from jax.experimental.pallas import tpu as pltpu
from jax.experimental.pallas import tpu_sc as plsc

NUM_EMBEDDINGS = 1000000
EMBEDDING_DIM = 64
PAD_DIM = 128
BATCH = 16384
HIST = 20

_info = plsc.get_sparse_core_info()
_NC, _NS = _info.num_cores, _info.num_subcores
_NW = _NC * _NS
_BBLKS = BATCH // 128
_BBLK_PER_W = _BBLKS // _NW
_UNITS = _BBLK_PER_W * HIST
_DBLK = EMBEDDING_DIM // 8


def _make_kernel():
    mesh = plsc.VectorSubcoreMesh(core_axis_name="c", subcore_axis_name="s")

    @functools.partial(
        pl.kernel,
        mesh=mesh,
        out_type=jax.ShapeDtypeStruct((HIST, _DBLK, _BBLKS, 8, 128),
                                      jnp.float32),
        compiler_params=pltpu.CompilerParams(use_tc_tiling_on_sc=False,
                                             needs_layout_passes=False),
        scratch_types=[
            pltpu.VMEM((HIST, 128 * _BBLK_PER_W), jnp.int32),
            pltpu.VMEM((128, PAD_DIM), jnp.float32),
            pltpu.VMEM((128, PAD_DIM), jnp.float32),
            pltpu.VMEM((128, PAD_DIM), jnp.float32),
            pltpu.VMEM((128, PAD_DIM), jnp.float32),
            pltpu.VMEM((_DBLK, 8, 128), jnp.float32),
            pltpu.VMEM((_DBLK, 8, 128), jnp.float32),
            pltpu.SemaphoreType.DMA,
            pltpu.SemaphoreType.DMA,
            pltpu.SemaphoreType.DMA,
            pltpu.SemaphoreType.DMA,
            pltpu.SemaphoreType.DMA,
            pltpu.SemaphoreType.DMA,
        ],
    )
    def gather_kernel(table_hbm, idxt_hbm, out_hbm, idx_v, rb0, rb1, rb2,
                      rb3, st0, st1, gs0, gs1, gs2, gs3, ws0, ws1):
        wid = lax.axis_index("s") * _NC + lax.axis_index("c")
        bblk0 = wid * _BBLK_PER_W

        pltpu.sync_copy(
            idxt_hbm.at[:, pl.ds(bblk0 * 128, 128 * _BBLK_PER_W)], idx_v)

        rbufs = (rb0, rb1, rb2, rb3)
        stages = (st0, st1)
        gsems = (gs0, gs1, gs2, gs3)
        wsems = (ws0, ws1)
        iota16 = lax.broadcasted_iota(jnp.int32, (16,), 0)

        def unit(g):
            return g % HIST, g // HIST

        def gather_src(g):
            h, bl = unit(g)
            return table_hbm.at[idx_v.at[h, pl.ds(bl * 128, 128)]]

        def out_dst(g):
            h, bl = unit(g)
            return out_hbm.at[h, :, bblk0 + bl, :, :]

        pltpu.async_copy(gather_src(0), rbufs[0], gsems[0])
        pltpu.async_copy(gather_src(1), rbufs[1], gsems[1])

        def body(i, carry):
            for j in range(4):
                g = 4 * i + j
                s = j % 2
                nxt = (j + 2) % 4

                @pl.when(g + 2 < _UNITS)
                def _():
                    pltpu.async_copy(gather_src(g + 2), rbufs[nxt],
                                     gsems[nxt])

                pltpu.make_async_copy(gather_src(g), rbufs[j],
                                      gsems[j]).wait()

                @pl.when(g >= 2)
                def _():
                    pltpu.make_async_copy(stages[s], out_dst(g - 2),
                                          wsems[s]).wait()

                def assemble(k8, c):
                    for ku in range(8):
                        dv = (k8 * 8 + ku + iota16) & (EMBEDDING_DIM - 1)
                        dblkv = dv >> 3
                        dsubv = dv & 7
                        for grp in range(8):
                            bv = grp * 16 + iota16
                            vals = plsc.load_gather(rbufs[j], [bv, dv])
                            plsc.store_scatter(stages[s],
                                               [dblkv, dsubv, bv], vals)
                    return c

                lax.fori_loop(0, EMBEDDING_DIM // 8, assemble, 0)

                pltpu.async_copy(stages[s], out_dst(g), wsems[s])
            return carry

        lax.fori_loop(0, _UNITS // 4, body, 0)
        for s in range(2):
            pltpu.make_async_copy(stages[s], out_dst(_UNITS - 2 + s),
                                  wsems[s]).wait()

    return gather_kernel


_gather = _make_kernel()


def kernel(token_ids, embedding):
    emb_pad = jnp.pad(embedding, ((0, 0), (0, PAD_DIM - EMBEDDING_DIM)))
    tok_t = token_ids.T
    out5 = _gather(emb_pad, tok_t)
    return out5.transpose(2, 4, 0, 1, 3).reshape(BATCH, HIST, EMBEDDING_DIM)


# speedup vs baseline: 1.0145x; 1.0145x over previous
import functools

import jax
import jax.numpy as jnp
from jax import lax
from jax.experimental import pallas as pl
from jax.experimental.pallas import tpu as pltpu
from jax.experimental.pallas import tpu_sc as plsc

NUM_EMBEDDINGS = 1000000
EMBEDDING_DIM = 64
PAD_DIM = 128
BATCH = 16384
HIST = 20

_info = plsc.get_sparse_core_info()
_NC, _NS = _info.num_cores, _info.num_subcores
_NW = _NC * _NS
_BBLKS = BATCH // 128
_BBLK_PER_W = _BBLKS // _NW
_UNITS = _BBLK_PER_W * HIST
_DBLK = EMBEDDING_DIM // 8


def _make_kernel():
    mesh = plsc.VectorSubcoreMesh(core_axis_name="c", subcore_axis_name="s")

    @functools.partial(
        pl.kernel,
        mesh=mesh,
        out_type=jax.ShapeDtypeStruct((HIST, _DBLK, _BBLKS, 8, 128),
                                      jnp.float32),
        compiler_params=pltpu.CompilerParams(use_tc_tiling_on_sc=False,
                                             needs_layout_passes=False),
        scratch_types=[
            pltpu.VMEM((HIST, 128 * _BBLK_PER_W), jnp.int32),
            pltpu.VMEM((128, PAD_DIM), jnp.float32),
            pltpu.VMEM((128, PAD_DIM), jnp.float32),
            pltpu.VMEM((_DBLK, 8, 128), jnp.float32),
            pltpu.VMEM((_DBLK, 8, 128), jnp.float32),
            pltpu.SemaphoreType.DMA,
            pltpu.SemaphoreType.DMA,
            pltpu.SemaphoreType.DMA,
            pltpu.SemaphoreType.DMA,
        ],
    )
    def gather_kernel(table_hbm, idxt_hbm, out_hbm, idx_v, rb0, rb1,
                      st0, st1, gs0, gs1, ws0, ws1):
        wid = lax.axis_index("s") * _NC + lax.axis_index("c")
        bblk0 = wid * _BBLK_PER_W

        pltpu.sync_copy(
            idxt_hbm.at[:, pl.ds(bblk0 * 128, 128 * _BBLK_PER_W)], idx_v)

        rbufs = (rb0, rb1)
        stages = (st0, st1)
        gsems = (gs0, gs1)
        wsems = (ws0, ws1)
        iota16 = lax.broadcasted_iota(jnp.int32, (16,), 0)

        def unit(g):
            return g % HIST, g // HIST

        def gather_src(g):
            h, bl = unit(g)
            return table_hbm.at[idx_v.at[h, pl.ds(bl * 128, 128)]]

        def out_dst(g):
            h, bl = unit(g)
            return out_hbm.at[h, :, bblk0 + bl, :, :]

        pltpu.async_copy(gather_src(0), rbufs[0], gsems[0])

        def body(i, carry):
            for j in range(2):
                g = 2 * i + j
                nxt = (j + 1) % 2

                @pl.when(g + 1 < _UNITS)
                def _():
                    pltpu.async_copy(gather_src(g + 1), rbufs[nxt],
                                     gsems[nxt])

                pltpu.make_async_copy(gather_src(g), rbufs[j],
                                      gsems[j]).wait()

                @pl.when(i > 0)
                def _():
                    pltpu.make_async_copy(stages[j], out_dst(g - 2),
                                          wsems[j]).wait()

                def assemble(k, c):
                    dv = (k + iota16) & (EMBEDDING_DIM - 1)
                    dblkv = dv >> 3
                    dsubv = dv & 7
                    for grp in range(8):
                        bv = grp * 16 + iota16
                        vals = plsc.load_gather(rbufs[j], [bv, dv])
                        plsc.store_scatter(stages[j], [dblkv, dsubv, bv],
                                           vals)
                    return c

                lax.fori_loop(0, EMBEDDING_DIM, assemble, 0)

                pltpu.async_copy(stages[j], out_dst(g), wsems[j])
            return carry

        lax.fori_loop(0, _UNITS // 2, body, 0)
        for j in range(2):
            pltpu.make_async_copy(stages[j], out_dst(_UNITS - 2 + j),
                                  wsems[j]).wait()

    return gather_kernel


_gather = _make_kernel()


def kernel(token_ids, embedding):
    emb_pad = jnp.pad(embedding, ((0, 0), (0, PAD_DIM - EMBEDDING_DIM)))
    tok_t = token_ids.T
    out5 = _gather(emb_pad, tok_t)
    return out5.transpose(2, 4, 0, 1, 3).reshape(BATCH, HIST, EMBEDDING_DIM)
